# baseline (speedup 1.0000x reference)
import functools

import jax
import jax.numpy as jnp
from jax import lax
from jax.experimental import pallas as pl
from jax.experimental.pallas import tpu as pltpu
from jax.experimental.pallas import tpu_sc as plsc

NC, NS, L = 2, 16, 16
NW = NC * NS


def _build(B, H2, D, SUB):
    nsub = (B * H2) // SUB
    spw = nsub // NW
    opw = B // NW
    rps = H2 // SUB
    nch = D // L
    mesh = plsc.VectorSubcoreMesh(core_axis_name="c", subcore_axis_name="s")

    @functools.partial(
        pl.kernel,
        out_type=jax.ShapeDtypeStruct((B, D), jnp.float32),
        mesh=mesh,
        scratch_types=[
            pltpu.VMEM((spw, SUB), jnp.int32),
            pltpu.VMEM((rps, SUB, D), jnp.float32),
            pltpu.VMEM((opw, D), jnp.float32),
            pltpu.VMEM((1, D), jnp.float32),
            pltpu.SemaphoreType.DMA,
        ],
        compiler_params=pltpu.CompilerParams(use_tc_tiling_on_sc=False),
    )
    def k(trees_hbm, table_hbm, out_hbm, idx_v, buf, out_v, t0_v, sem):
        wid = lax.axis_index("s") * NC + lax.axis_index("c")
        base_sub = wid * spw
        pltpu.sync_copy(trees_hbm.at[pl.ds(base_sub, spw)], idx_v)
        pltpu.sync_copy(table_hbm.at[pl.ds(0, 1)], t0_v)
        lanes = lax.iota(jnp.int32, L)
        rem = SUB - (SUB // L) * L

        def batch_body(b, _):
            s0 = b * rps
            cps = [
                pltpu.async_copy(table_hbm.at[idx_v.at[s0 + r]], buf.at[r], sem)
                for r in range(rps)
            ]
            one = jnp.ones((L,), jnp.int32)
            zero = jnp.zeros((L,), jnp.int32)
            cnt = zero
            for r in range(rps):
                for c in range(SUB // L):
                    v = idx_v[s0 + r, pl.ds(c * L, L)]
                    cnt = cnt + jnp.where(v == 0, one, zero)
                if rem:
                    v = idx_v[s0 + r, pl.ds(SUB - L, L)]
                    m = jnp.logical_and(v == 0, lanes >= (L - rem))
                    cnt = cnt + jnp.where(m, one, zero)
            dnums = lax.GatherDimensionNumbers(
                offset_dims=(), collapsed_slice_dims=(0,), start_index_map=(0,)
            )
            for s in (8, 4, 2, 1):
                perm = lax.gather(
                    cnt, (lanes ^ s)[:, None], dnums, (1,),
                    mode=lax.GatherScatterMode.PROMISE_IN_BOUNDS,
                )
                cnt = cnt + perm
            for cp in cps:
                cp.wait()

            def row_body(l, accs):
                new = []
                for c in range(nch):
                    a = accs[c]
                    for r in range(rps):
                        a = a + buf[r, l, pl.ds(c * L, L)]
                    new.append(a)
                return tuple(new)

            accs = lax.fori_loop(
                0, SUB, row_body,
                tuple(jnp.zeros((L,), jnp.float32) for _ in range(nch)),
            )
            cntf = cnt.astype(jnp.float32)
            for c in range(nch):
                out_v[b, pl.ds(c * L, L)] = (
                    accs[c] - cntf * t0_v[0, pl.ds(c * L, L)]
                )
            return 0

        lax.fori_loop(0, opw, batch_body, 0)
        pltpu.sync_copy(out_v, out_hbm.at[pl.ds(wid * opw, opw)])

    return k


@jax.jit
def kernel(trees, table):
    B, H = trees.shape
    _, D = table.shape
    SUB = 104
    H2 = 2 * SUB
    t = trees.astype(jnp.int32)
    t = jnp.pad(t, ((0, 0), (0, H2 - H)))
    t = t.reshape((B * H2) // SUB, SUB)
    return _build(B, H2, D, SUB)(t, table)
